# Initial kernel scaffold; baseline (speedup 1.0000x reference)
import jax
import jax.numpy as jnp
from jax.experimental import pallas as pl


def kernel(user_emb, item_emb, adj_row, adj_col, adj_val, W1, W2, W3):
    raise NotImplementedError("write your pallas kernel here")


# speedup vs baseline: 6.4673x; 6.4673x over previous
import functools

import jax
import jax.numpy as jnp
from jax import lax
from jax.experimental import pallas as pl
from jax.experimental.pallas import tpu as pltpu
from jax.experimental.pallas import tpu_sc as plsc

N_TOTAL = 10000
EMB = 128
N_EDGES = 320000
NC = 2
NS = 16
NW = NC * NS
EDGES_PER_TILE = N_EDGES // NW
CHUNK = 80
N_CHUNKS = EDGES_PER_TILE // CHUNK
ACC_ROWS = 10240
STRIPE = ACC_ROWS // NS
LANES = 16
GROUPS = EMB // LANES
ROW_BLOCK = 2000


def _spmm_sc(emb, row3d, col, val):
    mesh = plsc.VectorSubcoreMesh(core_axis_name="c", subcore_axis_name="s")
    out_sds = jax.ShapeDtypeStruct((N_TOTAL, EMB), jnp.float32)

    @functools.partial(
        pl.kernel,
        mesh=mesh,
        out_type=[out_sds, out_sds],
        scratch_types=[
            pltpu.VMEM((EDGES_PER_TILE,), jnp.int32),
            pltpu.VMEM((EDGES_PER_TILE,), jnp.float32),
            pltpu.VMEM((N_CHUNKS, CHUNK), jnp.int32),
            pltpu.VMEM((CHUNK, EMB), jnp.float32),
            pltpu.VMEM_SHARED((ACC_ROWS, EMB), jnp.float32),
            pltpu.SemaphoreType.DMA,
        ],
    )
    def k(emb_hbm, row_hbm, col_hbm, val_hbm, pa_hbm, pb_hbm,
          col_v, val_v, row_v, rows_v, acc, sem):
        cid = lax.axis_index("c")
        sid = lax.axis_index("s")
        wid = cid * NS + sid
        base = wid * EDGES_PER_TILE

        pltpu.sync_copy(col_hbm.at[pl.ds(base, EDGES_PER_TILE)], col_v)
        pltpu.sync_copy(val_hbm.at[pl.ds(base, EDGES_PER_TILE)], val_v)
        pltpu.sync_copy(row_hbm.at[wid], row_v)

        zero16 = jnp.zeros((LANES,), jnp.float32)

        @pl.loop(0, CHUNK)
        def _(r):
            for g in range(GROUPS):
                rows_v[r, pl.ds(g * LANES, LANES)] = zero16

        row0 = sid * STRIPE
        for z in range(STRIPE // CHUNK):
            pltpu.sync_copy(rows_v, acc.at[pl.ds(row0 + z * CHUNK, CHUNK)])
        plsc.subcore_barrier()

        @pl.loop(0, N_CHUNKS)
        def _(j):
            pltpu.sync_copy(emb_hbm.at[col_v.at[pl.ds(j * CHUNK, CHUNK)]],
                            rows_v)
            ebase = j * CHUNK

            @pl.loop(0, CHUNK, step=LANES)
            def _(c):
                vals16 = val_v[pl.ds(ebase + c, LANES)]
                for i in range(LANES):
                    vv = vals16.at[jnp.full((LANES,), i, jnp.int32)].get(
                        mode="promise_in_bounds")
                    for g in range(GROUPS):
                        sl = pl.ds(g * LANES, LANES)
                        rows_v[c + i, sl] = rows_v[c + i, sl] * vv

            pltpu.sync_copy(rows_v, acc.at[row_v.at[j]], add=True)

        plsc.subcore_barrier()

        def writeback(r0, n):
            @pl.when(cid == 0)
            def _():
                pltpu.sync_copy(acc.at[pl.ds(r0, n)], pa_hbm.at[pl.ds(r0, n)])

            @pl.when(cid == 1)
            def _():
                pltpu.sync_copy(acc.at[pl.ds(r0, n)], pb_hbm.at[pl.ds(r0, n)])

        @pl.when(sid < NS - 1)
        def _():
            writeback(row0, STRIPE)

        @pl.when(sid == NS - 1)
        def _():
            writeback((NS - 1) * STRIPE, N_TOTAL - (NS - 1) * STRIPE)

    return k(emb, row3d, col, val)


def _lrelu(x):
    return jnp.where(x >= 0, x, 0.3 * x)


def _gc_tc(pa, pb, W):
    def body(pa_ref, pb_ref, w_ref, e_ref, f_ref):
        pre = pa_ref[...] + pb_ref[...]
        e = lax.dot_general(pre, w_ref[...], (((1,), (1,)), ((), ())),
                            preferred_element_type=jnp.float32)
        e_ref[...] = e
        f_ref[...] = _lrelu(e)

    blk = lambda: pl.BlockSpec((ROW_BLOCK, EMB), lambda i: (i, 0))
    return pl.pallas_call(
        body,
        grid=(N_TOTAL // ROW_BLOCK,),
        in_specs=[blk(), blk(), pl.BlockSpec((EMB, EMB), lambda i: (0, 0))],
        out_specs=[blk(), blk()],
        out_shape=[jax.ShapeDtypeStruct((N_TOTAL, EMB), jnp.float32)] * 2,
    )(pa, pb, W)


def _final_tc(pa, pb, W, ego, e1, e3):
    def body(pa_ref, pb_ref, w_ref, ego_ref, e1_ref, e3_ref, out_ref):
        pre = pa_ref[...] + pb_ref[...]
        e5 = lax.dot_general(pre, w_ref[...], (((1,), (1,)), ((), ())),
                             preferred_element_type=jnp.float32)
        e1 = e1_ref[...]
        e3 = e3_ref[...]
        acc = ego_ref[...] + e1 + _lrelu(e1) + e3 + _lrelu(e3) + e5
        out_ref[...] = acc * (1.0 / 6.0)

    blk = lambda: pl.BlockSpec((ROW_BLOCK, EMB), lambda i: (i, 0))
    return pl.pallas_call(
        body,
        grid=(N_TOTAL // ROW_BLOCK,),
        in_specs=[blk(), blk(), pl.BlockSpec((EMB, EMB), lambda i: (0, 0)),
                  blk(), blk(), blk()],
        out_specs=blk(),
        out_shape=jax.ShapeDtypeStruct((N_TOTAL, EMB), jnp.float32),
    )(pa, pb, W, ego, e1, e3)


def kernel(user_emb, item_emb, adj_row, adj_col, adj_val, W1, W2, W3):
    n_users = user_emb.shape[0]
    ego = jnp.concatenate([user_emb, item_emb], axis=0)
    row3d = adj_row.astype(jnp.int32).reshape(NW, N_CHUNKS, CHUNK)
    col = adj_col.astype(jnp.int32)
    val = adj_val.astype(jnp.float32)

    pa, pb = _spmm_sc(ego, row3d, col, val)
    e1, f1 = _gc_tc(pa, pb, W1)
    pa, pb = _spmm_sc(f1, row3d, col, val)
    e3, f3 = _gc_tc(pa, pb, W2)
    pa, pb = _spmm_sc(f3, row3d, col, val)
    mean = _final_tc(pa, pb, W3, ego, e1, e3)
    return (mean[:n_users], mean[n_users:])


# speedup vs baseline: 9.7882x; 1.5135x over previous
import functools

import jax
import jax.numpy as jnp
from jax import lax
from jax.experimental import pallas as pl
from jax.experimental.pallas import tpu as pltpu
from jax.experimental.pallas import tpu_sc as plsc

N_TOTAL = 10000
EMB = 128
N_EDGES = 320000
NC = 2
NS = 16
NW = NC * NS
EDGES_PER_TILE = N_EDGES // NW
CHUNK = 80
N_CHUNKS = EDGES_PER_TILE // CHUNK
WIN = 25
NWIN = N_CHUNKS // WIN
WEDGES = WIN * CHUNK
ACC_ROWS = 10240
STRIPE = ACC_ROWS // NS
LANES = 16
GROUPS = EMB // LANES
ROW_BLOCK = 2000


def _spmm_sc(emb, row3d, col, val):
    mesh = plsc.VectorSubcoreMesh(core_axis_name="c", subcore_axis_name="s")
    out_sds = jax.ShapeDtypeStruct((N_TOTAL, EMB), jnp.float32)

    @functools.partial(
        pl.kernel,
        mesh=mesh,
        out_type=[out_sds, out_sds],
        scratch_types=[
            pltpu.VMEM((WEDGES,), jnp.int32),
            pltpu.VMEM((WEDGES,), jnp.int32),
            pltpu.VMEM((WEDGES,), jnp.float32),
            pltpu.VMEM((WEDGES,), jnp.float32),
            pltpu.VMEM((WIN, CHUNK), jnp.int32),
            pltpu.VMEM((WIN, CHUNK), jnp.int32),
            pltpu.VMEM((CHUNK, EMB), jnp.float32),
            pltpu.VMEM((CHUNK, EMB), jnp.float32),
            pltpu.VMEM_SHARED((ACC_ROWS, EMB), jnp.float32),
            pltpu.SemaphoreType.DMA,
            pltpu.SemaphoreType.DMA,
            pltpu.SemaphoreType.DMA,
            pltpu.SemaphoreType.DMA,
            pltpu.SemaphoreType.DMA,
            pltpu.SemaphoreType.DMA,
        ],
    )
    def k(emb_hbm, row_hbm, col_hbm, val_hbm, pa_hbm, pb_hbm,
          col_v0, col_v1, val_v0, val_v1, row_v0, row_v1, buf0, buf1, acc,
          gsem0, gsem1, ssem0, ssem1, stsem0, stsem1):
        cid = lax.axis_index("c")
        sid = lax.axis_index("s")
        wid = cid * NS + sid
        base = wid * EDGES_PER_TILE
        stsems = (stsem0, stsem1)
        col_bufs = (col_v0, col_v1)
        val_bufs = (val_v0, val_v1)
        row_bufs = (row_v0, row_v1)

        def stage_window(w):
            p = w % 2
            b = base + w * WEDGES
            return (
                pltpu.async_copy(col_hbm.at[pl.ds(b, WEDGES)],
                                 col_bufs[p], stsems[p]),
                pltpu.async_copy(val_hbm.at[pl.ds(b, WEDGES)],
                                 val_bufs[p], stsems[p]),
                pltpu.async_copy(row_hbm.at[wid, w], row_bufs[p], stsems[p]),
            )

        stage0 = stage_window(0)

        zero16 = jnp.zeros((LANES,), jnp.float32)

        @pl.loop(0, CHUNK)
        def _(r):
            for g in range(GROUPS):
                buf0[r, pl.ds(g * LANES, LANES)] = zero16

        row0 = sid * STRIPE
        for z in range(STRIPE // CHUNK):
            pltpu.sync_copy(buf0, acc.at[pl.ds(row0 + z * CHUNK, CHUNK)])
        for cpy in stage0:
            cpy.wait()
        plsc.subcore_barrier()

        def start_gather(p, kk, buf, sem):
            return pltpu.async_copy(
                emb_hbm.at[col_bufs[p].at[pl.ds(kk * CHUNK, CHUNK)]], buf, sem)

        def wait_gather(buf, sem):
            pltpu.make_async_copy(
                emb_hbm.at[col_v0.at[pl.ds(0, CHUNK)]], buf, sem).wait()

        def scale(buf, p, kk):
            ebase = kk * CHUNK

            @pl.loop(0, CHUNK, step=LANES)
            def _(c):
                vals16 = val_bufs[p][pl.ds(ebase + c, LANES)]
                for i in range(LANES):
                    vv = vals16.at[jnp.full((LANES,), i, jnp.int32)].get(
                        mode="promise_in_bounds")
                    for g in range(GROUPS):
                        sl = pl.ds(g * LANES, LANES)
                        buf[c + i, sl] = buf[c + i, sl] * vv

        def start_scatter(buf, p, kk, sem):
            return pltpu.async_copy(buf, acc.at[row_bufs[p].at[kk]], sem,
                                    add=True)

        def wait_scatter(buf, sem):
            pltpu.make_async_copy(buf, acc.at[row_v0.at[0]], sem).wait()

        start_gather(0, 0, buf0, gsem0)
        for w in range(NWIN):
            p = w % 2
            nxt = stage_window(w + 1) if w + 1 < NWIN else ()

            @pl.loop(0, WIN - 1, step=2)
            def _(kk):
                start_gather(p, kk + 1, buf1, gsem1)
                wait_gather(buf0, gsem0)
                scale(buf0, p, kk)
                start_scatter(buf0, p, kk, ssem0)
                wait_gather(buf1, gsem1)
                scale(buf1, p, kk + 1)
                start_scatter(buf1, p, kk + 1, ssem1)
                wait_scatter(buf0, ssem0)

                @pl.when(kk + 2 < WIN)
                def _():
                    start_gather(p, kk + 2, buf0, gsem0)

                wait_scatter(buf1, ssem1)

            wait_gather(buf0, gsem0)
            scale(buf0, p, WIN - 1)
            start_scatter(buf0, p, WIN - 1, ssem0)
            wait_scatter(buf0, ssem0)

            if w + 1 < NWIN:
                for cpy in nxt:
                    cpy.wait()
                start_gather(1 - p, 0, buf0, gsem0)

        plsc.subcore_barrier()

        def writeback(r0, n):
            @pl.when(cid == 0)
            def _():
                pltpu.sync_copy(acc.at[pl.ds(r0, n)], pa_hbm.at[pl.ds(r0, n)])

            @pl.when(cid == 1)
            def _():
                pltpu.sync_copy(acc.at[pl.ds(r0, n)], pb_hbm.at[pl.ds(r0, n)])

        @pl.when(sid < NS - 1)
        def _():
            writeback(row0, STRIPE)

        @pl.when(sid == NS - 1)
        def _():
            writeback((NS - 1) * STRIPE, N_TOTAL - (NS - 1) * STRIPE)

    return k(emb, row3d, col, val)


def _lrelu(x):
    return jnp.where(x >= 0, x, 0.3 * x)


def _gc_tc(pa, pb, W):
    def body(pa_ref, pb_ref, w_ref, e_ref, f_ref):
        pre = pa_ref[...] + pb_ref[...]
        e = lax.dot_general(pre, w_ref[...], (((1,), (1,)), ((), ())),
                            preferred_element_type=jnp.float32)
        e_ref[...] = e
        f_ref[...] = _lrelu(e)

    blk = lambda: pl.BlockSpec((ROW_BLOCK, EMB), lambda i: (i, 0))
    return pl.pallas_call(
        body,
        grid=(N_TOTAL // ROW_BLOCK,),
        in_specs=[blk(), blk(), pl.BlockSpec((EMB, EMB), lambda i: (0, 0))],
        out_specs=[blk(), blk()],
        out_shape=[jax.ShapeDtypeStruct((N_TOTAL, EMB), jnp.float32)] * 2,
    )(pa, pb, W)


def _final_tc(pa, pb, W, ego, e1, e3):
    def body(pa_ref, pb_ref, w_ref, ego_ref, e1_ref, e3_ref, out_ref):
        pre = pa_ref[...] + pb_ref[...]
        e5 = lax.dot_general(pre, w_ref[...], (((1,), (1,)), ((), ())),
                             preferred_element_type=jnp.float32)
        e1 = e1_ref[...]
        e3 = e3_ref[...]
        acc = ego_ref[...] + e1 + _lrelu(e1) + e3 + _lrelu(e3) + e5
        out_ref[...] = acc * (1.0 / 6.0)

    blk = lambda: pl.BlockSpec((ROW_BLOCK, EMB), lambda i: (i, 0))
    return pl.pallas_call(
        body,
        grid=(N_TOTAL // ROW_BLOCK,),
        in_specs=[blk(), blk(), pl.BlockSpec((EMB, EMB), lambda i: (0, 0)),
                  blk(), blk(), blk()],
        out_specs=blk(),
        out_shape=jax.ShapeDtypeStruct((N_TOTAL, EMB), jnp.float32),
    )(pa, pb, W, ego, e1, e3)


def kernel(user_emb, item_emb, adj_row, adj_col, adj_val, W1, W2, W3):
    n_users = user_emb.shape[0]
    ego = jnp.concatenate([user_emb, item_emb], axis=0)
    row3d = adj_row.astype(jnp.int32).reshape(NW, NWIN, WIN, CHUNK)
    col = adj_col.astype(jnp.int32)
    val = adj_val.astype(jnp.float32)

    pa, pb = _spmm_sc(ego, row3d, col, val)
    e1, f1 = _gc_tc(pa, pb, W1)
    pa, pb = _spmm_sc(f1, row3d, col, val)
    e3, f3 = _gc_tc(pa, pb, W2)
    pa, pb = _spmm_sc(f3, row3d, col, val)
    mean = _final_tc(pa, pb, W3, ego, e1, e3)
    return (mean[:n_users], mean[n_users:])


# speedup vs baseline: 12.3592x; 1.2627x over previous
import functools

import jax
import jax.numpy as jnp
from jax import lax
from jax.experimental import pallas as pl
from jax.experimental.pallas import tpu as pltpu
from jax.experimental.pallas import tpu_sc as plsc

N_TOTAL = 10000
EMB = 128
N_EDGES = 320000
NC = 2
NS = 16
NW = NC * NS
EDGES_PER_TILE = N_EDGES // NW
CHUNK = 80
N_CHUNKS = EDGES_PER_TILE // CHUNK
WIN = 25
NWIN = N_CHUNKS // WIN
WEDGES = WIN * CHUNK
ACC_ROWS = 10240
STRIPE = ACC_ROWS // NS
LANES = 16
GROUPS = EMB // LANES
ROW_BLOCK = 2000


def _spmm_sc(emb, row3d, col, val):
    mesh = plsc.VectorSubcoreMesh(core_axis_name="c", subcore_axis_name="s")
    out_sds = jax.ShapeDtypeStruct((N_TOTAL, EMB), jnp.float32)

    @functools.partial(
        pl.kernel,
        mesh=mesh,
        out_type=[out_sds, out_sds],
        scratch_types=[
            pltpu.VMEM((WEDGES,), jnp.int32),
            pltpu.VMEM((WEDGES,), jnp.int32),
            pltpu.VMEM((WEDGES,), jnp.float32),
            pltpu.VMEM((WEDGES,), jnp.float32),
            pltpu.VMEM((WIN, CHUNK), jnp.int32),
            pltpu.VMEM((WIN, CHUNK), jnp.int32),
            pltpu.VMEM((CHUNK, EMB), jnp.float32),
            pltpu.VMEM((CHUNK, EMB), jnp.float32),
            pltpu.VMEM((CHUNK, EMB), jnp.float32),
            pltpu.VMEM_SHARED((ACC_ROWS, EMB), jnp.float32),
            pltpu.SemaphoreType.DMA,
            pltpu.SemaphoreType.DMA,
            pltpu.SemaphoreType.DMA,
            pltpu.SemaphoreType.DMA,
            pltpu.SemaphoreType.DMA,
            pltpu.SemaphoreType.DMA,
            pltpu.SemaphoreType.DMA,
            pltpu.SemaphoreType.DMA,
        ],
    )
    def k(emb_hbm, row_hbm, col_hbm, val_hbm, pa_hbm, pb_hbm,
          col_v0, col_v1, val_v0, val_v1, row_v0, row_v1, buf0, buf1, buf2,
          acc, gsem0, gsem1, gsem2, ssem0, ssem1, ssem2, stsem0, stsem1):
        cid = lax.axis_index("c")
        sid = lax.axis_index("s")
        wid = cid * NS + sid
        base = wid * EDGES_PER_TILE
        stsems = (stsem0, stsem1)
        col_bufs = (col_v0, col_v1)
        val_bufs = (val_v0, val_v1)
        row_bufs = (row_v0, row_v1)
        bufs = (buf0, buf1, buf2)
        gsems = (gsem0, gsem1, gsem2)
        ssems = (ssem0, ssem1, ssem2)

        def stage_window(w):
            p = w % 2
            b = base + w * WEDGES
            return (
                pltpu.async_copy(col_hbm.at[pl.ds(b, WEDGES)],
                                 col_bufs[p], stsems[p]),
                pltpu.async_copy(val_hbm.at[pl.ds(b, WEDGES)],
                                 val_bufs[p], stsems[p]),
                pltpu.async_copy(row_hbm.at[wid, w], row_bufs[p], stsems[p]),
            )

        stage0 = stage_window(0)

        zero16 = jnp.zeros((LANES,), jnp.float32)

        @pl.loop(0, CHUNK)
        def _(r):
            for g in range(GROUPS):
                buf2[r, pl.ds(g * LANES, LANES)] = zero16

        row0 = sid * STRIPE
        for z in range(STRIPE // CHUNK):
            pltpu.sync_copy(buf2, acc.at[pl.ds(row0 + z * CHUNK, CHUNK)])
        for cpy in stage0:
            cpy.wait()
        plsc.subcore_barrier()

        def start_gather(p, kk, buf, sem):
            return pltpu.async_copy(
                emb_hbm.at[col_bufs[p].at[pl.ds(kk * CHUNK, CHUNK)]], buf, sem)

        def wait_gather(buf, sem):
            pltpu.make_async_copy(
                emb_hbm.at[col_v0.at[pl.ds(0, CHUNK)]], buf, sem).wait()

        def scale(buf, p, kk):
            ebase = kk * CHUNK

            @pl.loop(0, CHUNK, step=LANES)
            def _(c):
                vals16 = val_bufs[p][pl.ds(ebase + c, LANES)]
                for i in range(LANES):
                    vv = vals16.at[jnp.full((LANES,), i, jnp.int32)].get(
                        mode="promise_in_bounds")
                    for g in range(GROUPS):
                        sl = pl.ds(g * LANES, LANES)
                        buf[c + i, sl] = buf[c + i, sl] * vv

        def start_scatter(buf, p, kk, sem):
            return pltpu.async_copy(buf, acc.at[row_bufs[p].at[kk]], sem,
                                    add=True)

        def wait_scatter(buf, sem):
            pltpu.make_async_copy(buf, acc.at[row_v0.at[0]], sem).wait()

        start_gather(0, 0, bufs[0], gsems[0])
        start_gather(0, 1, bufs[1], gsems[1])
        start_scatter(bufs[2], 0, 0, ssems[2])

        def step(o, p, c, j, nxt_gather):
            a = (o + j) % 3
            nb = (o + j + 2) % 3
            wait_gather(bufs[a], gsems[a])
            wait_scatter(bufs[nb], ssems[nb])
            nxt_gather(bufs[nb], gsems[nb])
            scale(bufs[a], p, c)
            start_scatter(bufs[a], p, c, ssems[a])

        for w in range(NWIN):
            o = w % 3
            p = w % 2
            nxt = stage_window(w + 1) if w + 1 < NWIN else ()

            @pl.loop(0, WIN - 4, step=3)
            def _(kk):
                for j in range(3):
                    step(o, p, kk + j, j,
                         lambda buf, sem, jj=j: start_gather(
                             p, kk + jj + 2, buf, sem))

            for c in range(WIN - 4, WIN):
                j = c % 3
                if c + 2 < WIN:
                    nxt_gather = (lambda buf, sem, cc=c + 2:
                                  start_gather(p, cc, buf, sem))
                elif w + 1 < NWIN:
                    if c + 2 == WIN:
                        for cpy in nxt:
                            cpy.wait()
                    nxt_gather = (lambda buf, sem, cc=c + 2 - WIN:
                                  start_gather(1 - p, cc, buf, sem))
                else:
                    nxt_gather = lambda buf, sem: None
                step(o, p, c, j, nxt_gather)

            if w + 1 == NWIN:
                last = (o + WIN - 1) % 3
                wait_scatter(bufs[last], ssems[last])

        plsc.subcore_barrier()

        def writeback(r0, n):
            @pl.when(cid == 0)
            def _():
                pltpu.sync_copy(acc.at[pl.ds(r0, n)], pa_hbm.at[pl.ds(r0, n)])

            @pl.when(cid == 1)
            def _():
                pltpu.sync_copy(acc.at[pl.ds(r0, n)], pb_hbm.at[pl.ds(r0, n)])

        @pl.when(sid < NS - 1)
        def _():
            writeback(row0, STRIPE)

        @pl.when(sid == NS - 1)
        def _():
            writeback((NS - 1) * STRIPE, N_TOTAL - (NS - 1) * STRIPE)

    return k(emb, row3d, col, val)


def _lrelu(x):
    return jnp.where(x >= 0, x, 0.3 * x)


def _gc_tc(pa, pb, W):
    def body(pa_ref, pb_ref, w_ref, e_ref, f_ref):
        pre = pa_ref[...] + pb_ref[...]
        e = lax.dot_general(pre, w_ref[...], (((1,), (1,)), ((), ())),
                            preferred_element_type=jnp.float32)
        e_ref[...] = e
        f_ref[...] = _lrelu(e)

    blk = lambda: pl.BlockSpec((ROW_BLOCK, EMB), lambda i: (i, 0))
    return pl.pallas_call(
        body,
        grid=(N_TOTAL // ROW_BLOCK,),
        in_specs=[blk(), blk(), pl.BlockSpec((EMB, EMB), lambda i: (0, 0))],
        out_specs=[blk(), blk()],
        out_shape=[jax.ShapeDtypeStruct((N_TOTAL, EMB), jnp.float32)] * 2,
    )(pa, pb, W)


def _final_tc(pa, pb, W, ego, e1, e3):
    def body(pa_ref, pb_ref, w_ref, ego_ref, e1_ref, e3_ref, out_ref):
        pre = pa_ref[...] + pb_ref[...]
        e5 = lax.dot_general(pre, w_ref[...], (((1,), (1,)), ((), ())),
                             preferred_element_type=jnp.float32)
        e1 = e1_ref[...]
        e3 = e3_ref[...]
        acc = ego_ref[...] + e1 + _lrelu(e1) + e3 + _lrelu(e3) + e5
        out_ref[...] = acc * (1.0 / 6.0)

    blk = lambda: pl.BlockSpec((ROW_BLOCK, EMB), lambda i: (i, 0))
    return pl.pallas_call(
        body,
        grid=(N_TOTAL // ROW_BLOCK,),
        in_specs=[blk(), blk(), pl.BlockSpec((EMB, EMB), lambda i: (0, 0)),
                  blk(), blk(), blk()],
        out_specs=blk(),
        out_shape=jax.ShapeDtypeStruct((N_TOTAL, EMB), jnp.float32),
    )(pa, pb, W, ego, e1, e3)


def kernel(user_emb, item_emb, adj_row, adj_col, adj_val, W1, W2, W3):
    n_users = user_emb.shape[0]
    ego = jnp.concatenate([user_emb, item_emb], axis=0)
    row3d = adj_row.astype(jnp.int32).reshape(NW, NWIN, WIN, CHUNK)
    col = adj_col.astype(jnp.int32)
    val = adj_val.astype(jnp.float32)

    pa, pb = _spmm_sc(ego, row3d, col, val)
    e1, f1 = _gc_tc(pa, pb, W1)
    pa, pb = _spmm_sc(f1, row3d, col, val)
    e3, f3 = _gc_tc(pa, pb, W2)
    pa, pb = _spmm_sc(f3, row3d, col, val)
    mean = _final_tc(pa, pb, W3, ego, e1, e3)
    return (mean[:n_users], mean[n_users:])


# speedup vs baseline: 12.6889x; 1.0267x over previous
import functools

import jax
import jax.numpy as jnp
from jax import lax
from jax.experimental import pallas as pl
from jax.experimental.pallas import tpu as pltpu
from jax.experimental.pallas import tpu_sc as plsc

N_TOTAL = 10000
EMB = 128
N_EDGES = 320000
NC = 2
NS = 16
NW = NC * NS
EDGES_PER_TILE = N_EDGES // NW
CHUNK = 80
N_CHUNKS = EDGES_PER_TILE // CHUNK
ND = 4
NI = 8
ACC_ROWS = 10240
STRIPE = ACC_ROWS // NS
LANES = 16
GROUPS = EMB // LANES
ROW_BLOCK = 2000


def _spmm_sc(emb, row, col, val):
    mesh = plsc.VectorSubcoreMesh(core_axis_name="c", subcore_axis_name="s")
    out_sds = jax.ShapeDtypeStruct((N_TOTAL, EMB), jnp.float32)

    scratch = (
        [pltpu.VMEM((CHUNK,), jnp.int32) for _ in range(NI)]
        + [pltpu.VMEM((CHUNK,), jnp.float32) for _ in range(NI)]
        + [pltpu.VMEM((CHUNK,), jnp.int32) for _ in range(NI)]
        + [pltpu.VMEM((CHUNK, EMB), jnp.float32) for _ in range(ND)]
        + [pltpu.VMEM_SHARED((ACC_ROWS, EMB), jnp.float32)]
        + [pltpu.SemaphoreType.DMA] * (2 * ND + NI)
    )

    @functools.partial(
        pl.kernel,
        mesh=mesh,
        out_type=[out_sds, out_sds],
        scratch_types=scratch,
    )
    def k(emb_hbm, row_hbm, col_hbm, val_hbm, pa_hbm, pb_hbm, *s):
        icol = s[0:NI]
        ival = s[NI:2 * NI]
        irow = s[2 * NI:3 * NI]
        dbufs = s[3 * NI:3 * NI + ND]
        acc = s[3 * NI + ND]
        gsems = s[3 * NI + ND + 1:3 * NI + ND + 1 + ND]
        ssems = s[3 * NI + 2 * ND + 1:3 * NI + 3 * ND + 1]
        isems = s[3 * NI + 3 * ND + 1:]
        cid = lax.axis_index("c")
        sid = lax.axis_index("s")
        wid = cid * NS + sid
        base = wid * EDGES_PER_TILE

        def start_istage(cidx, m):
            b = base + cidx * CHUNK
            return (
                pltpu.async_copy(col_hbm.at[pl.ds(b, CHUNK)], icol[m],
                                 isems[m]),
                pltpu.async_copy(val_hbm.at[pl.ds(b, CHUNK)], ival[m],
                                 isems[m]),
                pltpu.async_copy(row_hbm.at[pl.ds(b, CHUNK)], irow[m],
                                 isems[m]),
            )

        def wait_istage(m):
            pltpu.make_async_copy(col_hbm.at[pl.ds(0, CHUNK)], icol[m],
                                  isems[m]).wait()
            pltpu.make_async_copy(val_hbm.at[pl.ds(0, CHUNK)], ival[m],
                                  isems[m]).wait()
            pltpu.make_async_copy(row_hbm.at[pl.ds(0, CHUNK)], irow[m],
                                  isems[m]).wait()

        def start_gather(m, buf, sem):
            return pltpu.async_copy(
                emb_hbm.at[icol[m].at[pl.ds(0, CHUNK)]], buf, sem)

        def wait_gather(buf, sem):
            pltpu.make_async_copy(
                emb_hbm.at[icol[0].at[pl.ds(0, CHUNK)]], buf, sem).wait()

        def scale(buf, vref):
            @pl.loop(0, CHUNK, step=LANES)
            def _(c):
                vals16 = vref[pl.ds(c, LANES)]
                for i in range(LANES):
                    vv = vals16.at[jnp.full((LANES,), i, jnp.int32)].get(
                        mode="promise_in_bounds")
                    for g in range(GROUPS):
                        sl = pl.ds(g * LANES, LANES)
                        buf[c + i, sl] = buf[c + i, sl] * vv

        def start_scatter(buf, m, sem):
            return pltpu.async_copy(buf, acc.at[irow[m].at[pl.ds(0, CHUNK)]],
                                    sem, add=True)

        def wait_scatter(buf, sem):
            pltpu.make_async_copy(buf, acc.at[irow[0].at[pl.ds(0, CHUNK)]],
                                  sem).wait()

        for m in range(6):
            start_istage(m, m)
        zero16 = jnp.zeros((LANES,), jnp.float32)

        @pl.loop(0, CHUNK)
        def _(r):
            for g in range(GROUPS):
                dbufs[3][r, pl.ds(g * LANES, LANES)] = zero16

        row0 = sid * STRIPE
        for z in range(STRIPE // CHUNK):
            pltpu.sync_copy(dbufs[3], acc.at[pl.ds(row0 + z * CHUNK, CHUNK)])
        for m in range(3):
            wait_istage(m)
            start_gather(m, dbufs[m], gsems[m])
        plsc.subcore_barrier()
        start_scatter(dbufs[3], 0, ssems[3])

        def step(c, j, rolled):
            dj = j % ND
            nd = (dj + 3) % ND
            wait_gather(dbufs[dj], gsems[dj])
            wait_scatter(dbufs[nd], ssems[nd])
            if rolled and j == NI - 1:
                @pl.when(c + 6 < N_CHUNKS)
                def _():
                    for cpy in start_istage(c + 6, (j + 6) % NI):
                        pass
            elif rolled or c + 6 < N_CHUNKS:
                start_istage(c + 6, (j + 6) % NI)
            if rolled or c + 3 < N_CHUNKS:
                wait_istage((j + 3) % NI)
                start_gather((j + 3) % NI, dbufs[nd], gsems[nd])
            scale(dbufs[dj], ival[j])
            start_scatter(dbufs[dj], j, ssems[dj])

        @pl.loop(0, N_CHUNKS - 5, step=NI)
        def _(kk):
            for j in range(NI):
                step(kk + j, j, True)

        for c in range(N_CHUNKS - 5, N_CHUNKS):
            step(c, c % NI, False)
        wait_scatter(dbufs[(N_CHUNKS - 1) % ND], ssems[(N_CHUNKS - 1) % ND])

        plsc.subcore_barrier()

        def writeback(r0, n):
            @pl.when(cid == 0)
            def _():
                pltpu.sync_copy(acc.at[pl.ds(r0, n)], pa_hbm.at[pl.ds(r0, n)])

            @pl.when(cid == 1)
            def _():
                pltpu.sync_copy(acc.at[pl.ds(r0, n)], pb_hbm.at[pl.ds(r0, n)])

        @pl.when(sid < NS - 1)
        def _():
            writeback(row0, STRIPE)

        @pl.when(sid == NS - 1)
        def _():
            writeback((NS - 1) * STRIPE, N_TOTAL - (NS - 1) * STRIPE)

    return k(emb, row, col, val)


def _lrelu(x):
    return jnp.where(x >= 0, x, 0.3 * x)


def _gc_tc(pa, pb, W):
    def body(pa_ref, pb_ref, w_ref, e_ref, f_ref):
        pre = pa_ref[...] + pb_ref[...]
        e = lax.dot_general(pre, w_ref[...], (((1,), (1,)), ((), ())),
                            preferred_element_type=jnp.float32)
        e_ref[...] = e
        f_ref[...] = _lrelu(e)

    blk = lambda: pl.BlockSpec((ROW_BLOCK, EMB), lambda i: (i, 0))
    return pl.pallas_call(
        body,
        grid=(N_TOTAL // ROW_BLOCK,),
        in_specs=[blk(), blk(), pl.BlockSpec((EMB, EMB), lambda i: (0, 0))],
        out_specs=[blk(), blk()],
        out_shape=[jax.ShapeDtypeStruct((N_TOTAL, EMB), jnp.float32)] * 2,
    )(pa, pb, W)


def _final_tc(pa, pb, W, ego, e1, e3):
    def body(pa_ref, pb_ref, w_ref, ego_ref, e1_ref, e3_ref, out_ref):
        pre = pa_ref[...] + pb_ref[...]
        e5 = lax.dot_general(pre, w_ref[...], (((1,), (1,)), ((), ())),
                             preferred_element_type=jnp.float32)
        e1 = e1_ref[...]
        e3 = e3_ref[...]
        acc = ego_ref[...] + e1 + _lrelu(e1) + e3 + _lrelu(e3) + e5
        out_ref[...] = acc * (1.0 / 6.0)

    blk = lambda: pl.BlockSpec((ROW_BLOCK, EMB), lambda i: (i, 0))
    return pl.pallas_call(
        body,
        grid=(N_TOTAL // ROW_BLOCK,),
        in_specs=[blk(), blk(), pl.BlockSpec((EMB, EMB), lambda i: (0, 0)),
                  blk(), blk(), blk()],
        out_specs=blk(),
        out_shape=jax.ShapeDtypeStruct((N_TOTAL, EMB), jnp.float32),
    )(pa, pb, W, ego, e1, e3)


def kernel(user_emb, item_emb, adj_row, adj_col, adj_val, W1, W2, W3):
    n_users = user_emb.shape[0]
    ego = jnp.concatenate([user_emb, item_emb], axis=0)
    row = adj_row.astype(jnp.int32)
    col = adj_col.astype(jnp.int32)
    val = adj_val.astype(jnp.float32)

    pa, pb = _spmm_sc(ego, row, col, val)
    e1, f1 = _gc_tc(pa, pb, W1)
    pa, pb = _spmm_sc(f1, row, col, val)
    e3, f3 = _gc_tc(pa, pb, W2)
    pa, pb = _spmm_sc(f3, row, col, val)
    mean = _final_tc(pa, pb, W3, ego, e1, e3)
    return (mean[:n_users], mean[n_users:])


# speedup vs baseline: 12.6988x; 1.0008x over previous
import functools

import jax
import jax.numpy as jnp
from jax import lax
from jax.experimental import pallas as pl
from jax.experimental.pallas import tpu as pltpu
from jax.experimental.pallas import tpu_sc as plsc

N_TOTAL = 10000
EMB = 128
N_EDGES = 320000
NC = 2
NS = 16
NW = NC * NS
EDGES_PER_TILE = N_EDGES // NW
CHUNK = 80
N_CHUNKS = EDGES_PER_TILE // CHUNK
ND = 4
NI = 8
ACC_ROWS = 10240
STRIPE = ACC_ROWS // NS
LANES = 16
GROUPS = EMB // LANES
ROW_BLOCK = 10000


def _spmm_sc(emb, row, col, val):
    mesh = plsc.VectorSubcoreMesh(core_axis_name="c", subcore_axis_name="s")
    out_sds = jax.ShapeDtypeStruct((N_TOTAL, EMB), jnp.float32)

    scratch = (
        [pltpu.VMEM((CHUNK,), jnp.int32) for _ in range(NI)]
        + [pltpu.VMEM((CHUNK,), jnp.float32) for _ in range(NI)]
        + [pltpu.VMEM((CHUNK,), jnp.int32) for _ in range(NI)]
        + [pltpu.VMEM((CHUNK, EMB), jnp.float32) for _ in range(ND)]
        + [pltpu.VMEM_SHARED((ACC_ROWS, EMB), jnp.float32)]
        + [pltpu.SemaphoreType.DMA] * (2 * ND + NI)
    )

    @functools.partial(
        pl.kernel,
        mesh=mesh,
        out_type=[out_sds, out_sds],
        scratch_types=scratch,
    )
    def k(emb_hbm, row_hbm, col_hbm, val_hbm, pa_hbm, pb_hbm, *s):
        icol = s[0:NI]
        ival = s[NI:2 * NI]
        irow = s[2 * NI:3 * NI]
        dbufs = s[3 * NI:3 * NI + ND]
        acc = s[3 * NI + ND]
        gsems = s[3 * NI + ND + 1:3 * NI + ND + 1 + ND]
        ssems = s[3 * NI + 2 * ND + 1:3 * NI + 3 * ND + 1]
        isems = s[3 * NI + 3 * ND + 1:]
        cid = lax.axis_index("c")
        sid = lax.axis_index("s")
        wid = cid * NS + sid
        base = wid * EDGES_PER_TILE

        def start_istage(cidx, m):
            b = base + cidx * CHUNK
            return (
                pltpu.async_copy(col_hbm.at[pl.ds(b, CHUNK)], icol[m],
                                 isems[m]),
                pltpu.async_copy(val_hbm.at[pl.ds(b, CHUNK)], ival[m],
                                 isems[m]),
                pltpu.async_copy(row_hbm.at[pl.ds(b, CHUNK)], irow[m],
                                 isems[m]),
            )

        def wait_istage(m):
            pltpu.make_async_copy(col_hbm.at[pl.ds(0, CHUNK)], icol[m],
                                  isems[m]).wait()
            pltpu.make_async_copy(val_hbm.at[pl.ds(0, CHUNK)], ival[m],
                                  isems[m]).wait()
            pltpu.make_async_copy(row_hbm.at[pl.ds(0, CHUNK)], irow[m],
                                  isems[m]).wait()

        def start_gather(m, buf, sem):
            return pltpu.async_copy(
                emb_hbm.at[icol[m].at[pl.ds(0, CHUNK)]], buf, sem)

        def wait_gather(buf, sem):
            pltpu.make_async_copy(
                emb_hbm.at[icol[0].at[pl.ds(0, CHUNK)]], buf, sem).wait()

        def scale(buf, vref):
            @pl.loop(0, CHUNK, step=LANES)
            def _(c):
                vals16 = vref[pl.ds(c, LANES)]
                for i in range(LANES):
                    vv = vals16.at[jnp.full((LANES,), i, jnp.int32)].get(
                        mode="promise_in_bounds")
                    for g in range(GROUPS):
                        sl = pl.ds(g * LANES, LANES)
                        buf[c + i, sl] = buf[c + i, sl] * vv

        def start_scatter(buf, m, sem):
            return pltpu.async_copy(buf, acc.at[irow[m].at[pl.ds(0, CHUNK)]],
                                    sem, add=True)

        def wait_scatter(buf, sem):
            pltpu.make_async_copy(buf, acc.at[irow[0].at[pl.ds(0, CHUNK)]],
                                  sem).wait()

        for m in range(6):
            start_istage(m, m)
        zero16 = jnp.zeros((LANES,), jnp.float32)

        @pl.loop(0, CHUNK)
        def _(r):
            for g in range(GROUPS):
                dbufs[3][r, pl.ds(g * LANES, LANES)] = zero16

        row0 = sid * STRIPE
        for z in range(STRIPE // CHUNK):
            pltpu.sync_copy(dbufs[3], acc.at[pl.ds(row0 + z * CHUNK, CHUNK)])
        for m in range(3):
            wait_istage(m)
            start_gather(m, dbufs[m], gsems[m])
        plsc.subcore_barrier()
        start_scatter(dbufs[3], 0, ssems[3])

        def step(c, j, rolled):
            dj = j % ND
            nd = (dj + 3) % ND
            wait_gather(dbufs[dj], gsems[dj])
            wait_scatter(dbufs[nd], ssems[nd])
            if rolled and j == NI - 1:
                @pl.when(c + 6 < N_CHUNKS)
                def _():
                    for cpy in start_istage(c + 6, (j + 6) % NI):
                        pass
            elif rolled or c + 6 < N_CHUNKS:
                start_istage(c + 6, (j + 6) % NI)
            if rolled or c + 3 < N_CHUNKS:
                wait_istage((j + 3) % NI)
                start_gather((j + 3) % NI, dbufs[nd], gsems[nd])
            scale(dbufs[dj], ival[j])
            start_scatter(dbufs[dj], j, ssems[dj])

        @pl.loop(0, N_CHUNKS - 5, step=NI)
        def _(kk):
            for j in range(NI):
                step(kk + j, j, True)

        for c in range(N_CHUNKS - 5, N_CHUNKS):
            step(c, c % NI, False)
        wait_scatter(dbufs[(N_CHUNKS - 1) % ND], ssems[(N_CHUNKS - 1) % ND])

        plsc.subcore_barrier()

        def writeback(r0, n):
            @pl.when(cid == 0)
            def _():
                pltpu.sync_copy(acc.at[pl.ds(r0, n)], pa_hbm.at[pl.ds(r0, n)])

            @pl.when(cid == 1)
            def _():
                pltpu.sync_copy(acc.at[pl.ds(r0, n)], pb_hbm.at[pl.ds(r0, n)])

        @pl.when(sid < NS - 1)
        def _():
            writeback(row0, STRIPE)

        @pl.when(sid == NS - 1)
        def _():
            writeback((NS - 1) * STRIPE, N_TOTAL - (NS - 1) * STRIPE)

    return k(emb, row, col, val)


def _lrelu(x):
    return jnp.where(x >= 0, x, 0.3 * x)


def _gc_tc(pa, pb, W):
    def body(pa_ref, pb_ref, w_ref, e_ref, f_ref):
        pre = pa_ref[...] + pb_ref[...]
        e = lax.dot_general(pre, w_ref[...], (((1,), (1,)), ((), ())),
                            preferred_element_type=jnp.float32)
        e_ref[...] = e
        f_ref[...] = _lrelu(e)

    blk = lambda: pl.BlockSpec((ROW_BLOCK, EMB), lambda i: (i, 0))
    return pl.pallas_call(
        body,
        grid=(N_TOTAL // ROW_BLOCK,),
        in_specs=[blk(), blk(), pl.BlockSpec((EMB, EMB), lambda i: (0, 0))],
        out_specs=[blk(), blk()],
        out_shape=[jax.ShapeDtypeStruct((N_TOTAL, EMB), jnp.float32)] * 2,
    )(pa, pb, W)


def _final_tc(pa, pb, W, ego, e1, e3):
    def body(pa_ref, pb_ref, w_ref, ego_ref, e1_ref, e3_ref, out_ref):
        pre = pa_ref[...] + pb_ref[...]
        e5 = lax.dot_general(pre, w_ref[...], (((1,), (1,)), ((), ())),
                             preferred_element_type=jnp.float32)
        e1 = e1_ref[...]
        e3 = e3_ref[...]
        acc = ego_ref[...] + e1 + _lrelu(e1) + e3 + _lrelu(e3) + e5
        out_ref[...] = acc * (1.0 / 6.0)

    blk = lambda: pl.BlockSpec((ROW_BLOCK, EMB), lambda i: (i, 0))
    return pl.pallas_call(
        body,
        grid=(N_TOTAL // ROW_BLOCK,),
        in_specs=[blk(), blk(), pl.BlockSpec((EMB, EMB), lambda i: (0, 0)),
                  blk(), blk(), blk()],
        out_specs=blk(),
        out_shape=jax.ShapeDtypeStruct((N_TOTAL, EMB), jnp.float32),
    )(pa, pb, W, ego, e1, e3)


def kernel(user_emb, item_emb, adj_row, adj_col, adj_val, W1, W2, W3):
    n_users = user_emb.shape[0]
    ego = jnp.concatenate([user_emb, item_emb], axis=0)
    row = adj_row.astype(jnp.int32)
    col = adj_col.astype(jnp.int32)
    val = adj_val.astype(jnp.float32)

    pa, pb = _spmm_sc(ego, row, col, val)
    e1, f1 = _gc_tc(pa, pb, W1)
    pa, pb = _spmm_sc(f1, row, col, val)
    e3, f3 = _gc_tc(pa, pb, W2)
    pa, pb = _spmm_sc(f3, row, col, val)
    mean = _final_tc(pa, pb, W3, ego, e1, e3)
    return (mean[:n_users], mean[n_users:])
